# Initial kernel scaffold; baseline (speedup 1.0000x reference)
import jax
import jax.numpy as jnp
from jax.experimental import pallas as pl


def kernel(tile_features, params, hex_to_vertex, vertex_to_hex, edge_to_vertex, vertex_to_edge, h2v_mask, v2h_mask, e2v_mask, v2e_mask):
    raise NotImplementedError("write your pallas kernel here")


# speedup vs baseline: 1.7207x; 1.7207x over previous
import jax
import jax.numpy as jnp
from jax.experimental import pallas as pl
from jax.experimental.pallas import tpu as pltpu

TILE_IN = 20
HID = 64
OUT = 64
N_ROUNDS = 2
N_HEXES = 19
N_VERTICES = 54
N_EDGES = 72

_TB = 256


def _avg_mat(adj, mask, n_src):
    oh = (adj[..., None] == jnp.arange(n_src)[None, None, :]).astype(jnp.float32)
    m = mask.astype(jnp.float32)
    a = jnp.sum(oh * m[..., None], axis=1)
    cnt = jnp.clip(jnp.sum(m, axis=1), 1.0, None)
    return a / cnt[:, None]


def _mlp2d(x, w, b, g, beta):
    y = jnp.dot(x, w, preferred_element_type=jnp.float32) + b[None, :]
    mu = jnp.mean(y, axis=-1, keepdims=True)
    var = jnp.mean((y - mu) ** 2, axis=-1, keepdims=True)
    y = (y - mu) * jax.lax.rsqrt(var + 1e-5) * g[None, :] + beta[None, :]
    return jnp.maximum(y, 0.0)


def _gather3(a_ref, x3):
    return jax.lax.dot_general(a_ref[...], x3, (((1,), (0,)), ((), ())),
                               preferred_element_type=jnp.float32)


def _body(tf_ref, avh_ref, aev_ref, ahv_ref, ave_ref, *rest):
    w_refs = rest[:-1]
    out_ref = rest[-1]
    w = [r[...] for r in w_refs]
    (hw, hb, hg, hbt, vw, vb, vg, vbt, ew, eb, eg, ebt) = w[:12]
    hup = [w[12 + 4 * r:16 + 4 * r] for r in range(N_ROUNDS)]
    vup = [w[20 + 4 * r:24 + 4 * r] for r in range(N_ROUNDS)]
    eup = [w[28 + 4 * r:32 + 4 * r] for r in range(N_ROUNDS)]
    row, rob, rog, robt = w[36:40]

    t3 = tf_ref[...]
    tb = t3.shape[1]

    hex_h = _mlp2d(t3.reshape(N_HEXES * tb, TILE_IN), hw, hb, hg, hbt)
    hex_h = hex_h.reshape(N_HEXES, tb, HID)

    vraw = _gather3(avh_ref, t3)
    vertex_h = _mlp2d(vraw.reshape(N_VERTICES * tb, TILE_IN), vw, vb, vg, vbt)
    vertex_h = vertex_h.reshape(N_VERTICES, tb, HID)

    eraw = _gather3(aev_ref, vraw)
    edge_h = _mlp2d(eraw.reshape(N_EDGES * tb, TILE_IN), ew, eb, eg, ebt)
    edge_h = edge_h.reshape(N_EDGES, tb, HID)

    for r in range(N_ROUNDS):
        v_from_h = _gather3(avh_ref, hex_h)
        v_from_e = _gather3(ave_ref, edge_h)
        h_from_v = _gather3(ahv_ref, vertex_h)
        e_from_v = _gather3(aev_ref, vertex_h)
        hex_h = _mlp2d(
            jnp.concatenate([hex_h, h_from_v], axis=-1).reshape(N_HEXES * tb, 2 * HID),
            *hup[r]).reshape(N_HEXES, tb, HID)
        vertex_h = _mlp2d(
            jnp.concatenate([vertex_h, v_from_h, v_from_e], axis=-1).reshape(N_VERTICES * tb, 3 * HID),
            *vup[r]).reshape(N_VERTICES, tb, HID)
        edge_h = _mlp2d(
            jnp.concatenate([edge_h, e_from_v], axis=-1).reshape(N_EDGES * tb, 2 * HID),
            *eup[r]).reshape(N_EDGES, tb, HID)

    pooled = jnp.concatenate([
        jnp.mean(hex_h, axis=0),
        jnp.mean(vertex_h, axis=0),
        jnp.mean(edge_h, axis=0),
    ], axis=-1)
    out_ref[...] = _mlp2d(pooled, row, rob, rog, robt)


def kernel(tile_features, params, hex_to_vertex, vertex_to_hex, edge_to_vertex,
           vertex_to_edge, h2v_mask, v2h_mask, e2v_mask, v2e_mask):
    b = tile_features.shape[0]
    a_vh = _avg_mat(vertex_to_hex, v2h_mask, N_HEXES)
    a_ev = _avg_mat(edge_to_vertex, e2v_mask, N_VERTICES)
    a_hv = _avg_mat(hex_to_vertex, h2v_mask, N_VERTICES)
    a_ve = _avg_mat(vertex_to_edge, v2e_mask, N_EDGES)

    tf_t = jnp.transpose(tile_features, (1, 0, 2))

    weights = []
    for name in ('hex_in', 'vertex_in', 'edge_in'):
        weights.extend(params[name])
    for name in ('hex_up', 'vertex_up', 'edge_up'):
        for r in range(N_ROUNDS):
            weights.extend(params[name][r])
    weights.extend(params['readout'])

    full = lambda arr: pl.BlockSpec(arr.shape, lambda i: (0,) * arr.ndim)
    in_specs = [
        pl.BlockSpec((N_HEXES, _TB, TILE_IN), lambda i: (0, i, 0)),
        full(a_vh), full(a_ev), full(a_hv), full(a_ve),
    ] + [full(w) for w in weights]

    return pl.pallas_call(
        _body,
        grid=(b // _TB,),
        in_specs=in_specs,
        out_specs=pl.BlockSpec((_TB, OUT), lambda i: (i, 0)),
        out_shape=jax.ShapeDtypeStruct((b, OUT), jnp.float32),
        compiler_params=pltpu.CompilerParams(
            dimension_semantics=("arbitrary",),
        ),
    )(tf_t, a_vh, a_ev, a_hv, a_ve, *weights)


# speedup vs baseline: 1.7696x; 1.0284x over previous
import jax
import jax.numpy as jnp
from jax.experimental import pallas as pl
from jax.experimental.pallas import tpu as pltpu

TILE_IN = 20
HID = 64
OUT = 64
N_ROUNDS = 2
N_HEXES = 19
N_VERTICES = 54
N_EDGES = 72

_TB = 256


def _avg_mat(adj, mask, n_src):
    oh = (adj[..., None] == jnp.arange(n_src)[None, None, :]).astype(jnp.float32)
    m = mask.astype(jnp.float32)
    a = jnp.sum(oh * m[..., None], axis=1)
    cnt = jnp.clip(jnp.sum(m, axis=1), 1.0, None)
    return (a / cnt[:, None]).astype(jnp.bfloat16)


def _mlp2d(x, w, b, g, beta, out_dtype=jnp.bfloat16):
    y = jnp.dot(x, w, preferred_element_type=jnp.float32) + b[None, :]
    mu = jnp.mean(y, axis=-1, keepdims=True)
    var = jnp.mean((y - mu) ** 2, axis=-1, keepdims=True)
    y = (y - mu) * jax.lax.rsqrt(var + 1e-5) * g[None, :] + beta[None, :]
    return jnp.maximum(y, 0.0).astype(out_dtype)


def _gather3(a_ref, x3):
    return jax.lax.dot_general(a_ref[...], x3, (((1,), (0,)), ((), ())),
                               preferred_element_type=jnp.float32).astype(jnp.bfloat16)


def _body(tf_ref, avh_ref, aev_ref, ahv_ref, ave_ref, *rest):
    w_refs = rest[:-1]
    out_ref = rest[-1]
    w = [r[...] for r in w_refs]
    (hw, hb, hg, hbt, vw, vb, vg, vbt, ew, eb, eg, ebt) = w[:12]
    hup = [w[12 + 4 * r:16 + 4 * r] for r in range(N_ROUNDS)]
    vup = [w[20 + 4 * r:24 + 4 * r] for r in range(N_ROUNDS)]
    eup = [w[28 + 4 * r:32 + 4 * r] for r in range(N_ROUNDS)]
    row, rob, rog, robt = w[36:40]

    t3 = tf_ref[...].astype(jnp.bfloat16)
    tb = t3.shape[1]

    hex_h = _mlp2d(t3.reshape(N_HEXES * tb, TILE_IN), hw, hb, hg, hbt)
    hex_h = hex_h.reshape(N_HEXES, tb, HID)

    vraw = _gather3(avh_ref, t3)
    vertex_h = _mlp2d(vraw.reshape(N_VERTICES * tb, TILE_IN), vw, vb, vg, vbt)
    vertex_h = vertex_h.reshape(N_VERTICES, tb, HID)

    eraw = _gather3(aev_ref, vraw)
    edge_h = _mlp2d(eraw.reshape(N_EDGES * tb, TILE_IN), ew, eb, eg, ebt)
    edge_h = edge_h.reshape(N_EDGES, tb, HID)

    for r in range(N_ROUNDS):
        v_from_h = _gather3(avh_ref, hex_h)
        v_from_e = _gather3(ave_ref, edge_h)
        h_from_v = _gather3(ahv_ref, vertex_h)
        e_from_v = _gather3(aev_ref, vertex_h)
        hex_h = _mlp2d(
            jnp.concatenate([hex_h, h_from_v], axis=-1).reshape(N_HEXES * tb, 2 * HID),
            *hup[r]).reshape(N_HEXES, tb, HID)
        vertex_h = _mlp2d(
            jnp.concatenate([vertex_h, v_from_h, v_from_e], axis=-1).reshape(N_VERTICES * tb, 3 * HID),
            *vup[r]).reshape(N_VERTICES, tb, HID)
        edge_h = _mlp2d(
            jnp.concatenate([edge_h, e_from_v], axis=-1).reshape(N_EDGES * tb, 2 * HID),
            *eup[r]).reshape(N_EDGES, tb, HID)

    pooled = jnp.concatenate([
        jnp.mean(hex_h.astype(jnp.float32), axis=0),
        jnp.mean(vertex_h.astype(jnp.float32), axis=0),
        jnp.mean(edge_h.astype(jnp.float32), axis=0),
    ], axis=-1).astype(jnp.bfloat16)
    out_ref[...] = _mlp2d(pooled, row, rob, rog, robt, out_dtype=jnp.float32)


def kernel(tile_features, params, hex_to_vertex, vertex_to_hex, edge_to_vertex,
           vertex_to_edge, h2v_mask, v2h_mask, e2v_mask, v2e_mask):
    b = tile_features.shape[0]
    a_vh = _avg_mat(vertex_to_hex, v2h_mask, N_HEXES)
    a_ev = _avg_mat(edge_to_vertex, e2v_mask, N_VERTICES)
    a_hv = _avg_mat(hex_to_vertex, h2v_mask, N_VERTICES)
    a_ve = _avg_mat(vertex_to_edge, v2e_mask, N_EDGES)

    tf_t = jnp.transpose(tile_features, (1, 0, 2))

    weights = []
    for name in ('hex_in', 'vertex_in', 'edge_in'):
        weights.extend(params[name])
    for name in ('hex_up', 'vertex_up', 'edge_up'):
        for r in range(N_ROUNDS):
            weights.extend(params[name][r])
    weights.extend(params['readout'])
    weights = [w.astype(jnp.bfloat16) if i % 4 == 0 else w
               for i, w in enumerate(weights)]

    full = lambda arr: pl.BlockSpec(arr.shape, lambda i: (0,) * arr.ndim)
    in_specs = [
        pl.BlockSpec((N_HEXES, _TB, TILE_IN), lambda i: (0, i, 0)),
        full(a_vh), full(a_ev), full(a_hv), full(a_ve),
    ] + [full(w) for w in weights]

    return pl.pallas_call(
        _body,
        grid=(b // _TB,),
        in_specs=in_specs,
        out_specs=pl.BlockSpec((_TB, OUT), lambda i: (i, 0)),
        out_shape=jax.ShapeDtypeStruct((b, OUT), jnp.float32),
        compiler_params=pltpu.CompilerParams(
            dimension_semantics=("arbitrary",),
        ),
    )(tf_t, a_vh, a_ev, a_hv, a_ve, *weights)
